# baseline (device time: 32151 ns/iter reference)
import jax
import jax.numpy as jnp
from jax import lax
from jax.experimental import pallas as pl
from jax.experimental.pallas import tpu as pltpu

N_DEV = 4
B_LOC = 2
SQ = 128
SKV = 128
HG = 4
GW = 4 * 64
D_MODEL = 512
DH = 64


def _block_mask():
    qi = lax.broadcasted_iota(jnp.int32, (SQ, SKV), 0) // 64
    kj = lax.broadcasted_iota(jnp.int32, (SQ, SKV), 1) // 64
    return (qi == kj) | (kj == 0) | (((qi + kj) % 3) == 0)


def _body(x_ref, wq_ref, k_hbm, v_hbm, wo_ref, out_ref,
          k_vmem, v_vmem, cwq, cwo, kv_sems, send_sems, recv_sems):
    my = lax.axis_index("i")
    left = lax.rem(my + N_DEV - 1, N_DEV)
    right = lax.rem(my + 1, N_DEV)

    kv_copies = []
    for gg in range(HG):
        for hh in range(4):
            kc = pltpu.make_async_copy(
                k_hbm.at[pl.ds(my * B_LOC, B_LOC), :, 4 * gg + hh, :],
                k_vmem.at[gg, :, hh], kv_sems.at[4 * gg + hh])
            vc = pltpu.make_async_copy(
                v_hbm.at[pl.ds(my * B_LOC, B_LOC), :, 4 * gg + hh, :],
                v_vmem.at[gg, :, hh], kv_sems.at[16 + 4 * gg + hh])
            kc.start()
            vc.start()
            kv_copies += [kc, vc]

    barrier = pltpu.get_barrier_semaphore()
    for nbr in (left, right):
        pl.semaphore_signal(barrier, inc=1, device_id=(nbr,),
                            device_id_type=pl.DeviceIdType.MESH)
    pl.semaphore_wait(barrier, 2)

    cwq[0] = wq_ref[...].astype(jnp.bfloat16)
    cwo[0] = wo_ref[...].astype(jnp.bfloat16)

    def rdma(src, dst, sem_idx, dev):
        return pltpu.make_async_remote_copy(
            src_ref=src, dst_ref=dst,
            send_sem=send_sems.at[sem_idx], recv_sem=recv_sems.at[sem_idx],
            device_id=(dev,), device_id_type=pl.DeviceIdType.MESH)

    p1 = [
        rdma(cwq.at[0], cwq.at[1], 0, right),
        rdma(cwo.at[0], cwo.at[1], 1, right),
        rdma(cwq.at[0], cwq.at[2], 2, left),
        rdma(cwo.at[0], cwo.at[2], 3, left),
    ]
    for r in p1:
        r.start()

    for c in kv_copies:
        c.wait()

    x2b = x_ref[...].reshape(B_LOC * SQ, D_MODEL).astype(jnp.bfloat16)
    mask = _block_mask()

    def slot_contrib(slot, g):
        wq_g = cwq[slot]
        wo_g = cwo[slot]
        q = jnp.dot(x2b, wq_g, preferred_element_type=jnp.float32)
        qb = q.astype(jnp.bfloat16)
        ctx_rows = []
        for b in range(B_LOC):
            heads = []
            for hh in range(HG):
                qbh = qb[b * SQ:(b + 1) * SQ, hh * DH:(hh + 1) * DH]
                kbh = k_vmem[g, b, hh].astype(jnp.bfloat16)
                vbh = v_vmem[g, b, hh].astype(jnp.bfloat16)
                s = lax.dot_general(
                    qbh, kbh, (((1,), (1,)), ((), ())),
                    preferred_element_type=jnp.float32) * 0.125
                s = jnp.where(mask, s, -1e9)
                w = jnp.exp(s - jnp.max(s, axis=-1, keepdims=True))
                w = w / jnp.sum(w, axis=-1, keepdims=True)
                heads.append(jnp.dot(w.astype(jnp.bfloat16), vbh,
                                     preferred_element_type=jnp.float32))
            ctx_rows.append(jnp.concatenate(heads, axis=1))
        ctx = jnp.concatenate(ctx_rows, axis=0).astype(jnp.bfloat16)
        return jnp.dot(ctx, wo_g, preferred_element_type=jnp.float32)

    acc = slot_contrib(0, my)

    for r in p1:
        r.wait_recv()

    p2 = [
        rdma(cwq.at[1, pl.ds(0, 256)], cwq.at[3, pl.ds(0, 256)], 4, right),
        rdma(cwo.at[1, pl.ds(0, 128)], cwo.at[3, pl.ds(0, 128)], 5, right),
        rdma(cwq.at[2, pl.ds(256, 256)], cwq.at[3, pl.ds(256, 256)], 6, left),
        rdma(cwo.at[2, pl.ds(128, 128)], cwo.at[3, pl.ds(128, 128)], 7, left),
    ]
    for r in p2:
        r.start()

    acc = acc + slot_contrib(1, lax.rem(my + N_DEV - 1, N_DEV))
    acc = acc + slot_contrib(2, lax.rem(my + 1, N_DEV))

    for r in p2:
        r.wait_recv()
    acc = acc + slot_contrib(3, lax.rem(my + 2, N_DEV))

    for r in p1 + p2:
        r.wait_send()

    out_ref[...] = acc.reshape(B_LOC, SQ, D_MODEL)


def kernel(x, Wq, K_ext, V_ext, Wo):
    return pl.pallas_call(
        _body,
        out_shape=jax.ShapeDtypeStruct((B_LOC, SQ, D_MODEL), jnp.float32),
        in_specs=[
            pl.BlockSpec(memory_space=pltpu.MemorySpace.VMEM),
            pl.BlockSpec(memory_space=pltpu.MemorySpace.VMEM),
            pl.BlockSpec(memory_space=pl.ANY),
            pl.BlockSpec(memory_space=pl.ANY),
            pl.BlockSpec(memory_space=pltpu.MemorySpace.VMEM),
        ],
        out_specs=pl.BlockSpec(memory_space=pltpu.MemorySpace.VMEM),
        scratch_shapes=[
            pltpu.VMEM((HG, B_LOC, 4, SKV, DH), jnp.float32),
            pltpu.VMEM((HG, B_LOC, 4, SKV, DH), jnp.float32),
            pltpu.VMEM((N_DEV, D_MODEL, GW), jnp.bfloat16),
            pltpu.VMEM((N_DEV, GW, D_MODEL), jnp.bfloat16),
            pltpu.SemaphoreType.DMA((32,)),
            pltpu.SemaphoreType.DMA((8,)),
            pltpu.SemaphoreType.DMA((8,)),
        ],
        compiler_params=pltpu.CompilerParams(collective_id=0),
    )(x, Wq, K_ext, V_ext, Wo)


# device time: 29803 ns/iter; 1.0788x vs baseline; 1.0788x over previous
import jax
import jax.numpy as jnp
from jax import lax
from jax.experimental import pallas as pl
from jax.experimental.pallas import tpu as pltpu

N_DEV = 4
B_LOC = 2
SQ = 128
SKV = 128
HG = 4
GW = 4 * 64
D_MODEL = 512
DH = 64


def _block_mask():
    qi = lax.broadcasted_iota(jnp.int32, (SQ, SKV), 0) // 64
    kj = lax.broadcasted_iota(jnp.int32, (SQ, SKV), 1) // 64
    return (qi == kj) | (kj == 0) | (((qi + kj) % 3) == 0)


def _body(x_ref, wq_ref, k_hbm, v_hbm, wo_ref, out_ref,
          k_raw, v_raw, k_mrg, v_mrg, cwq, cwo,
          kv_sems, send_sems, recv_sems):
    my = lax.axis_index("i")
    left = lax.rem(my + N_DEV - 1, N_DEV)
    right = lax.rem(my + 1, N_DEV)

    kc = pltpu.make_async_copy(
        k_hbm.at[pl.ds(my * B_LOC, B_LOC)], k_raw, kv_sems.at[0])
    vc = pltpu.make_async_copy(
        v_hbm.at[pl.ds(my * B_LOC, B_LOC)], v_raw, kv_sems.at[1])
    kc.start()
    vc.start()

    barrier = pltpu.get_barrier_semaphore()
    for nbr in (left, right):
        pl.semaphore_signal(barrier, inc=1, device_id=(nbr,),
                            device_id_type=pl.DeviceIdType.MESH)
    pl.semaphore_wait(barrier, 2)

    cwq[0] = wq_ref[...].astype(jnp.bfloat16)
    cwo[0] = wo_ref[...].astype(jnp.bfloat16)

    def rdma(src, dst, sem_idx, dev):
        return pltpu.make_async_remote_copy(
            src_ref=src, dst_ref=dst,
            send_sem=send_sems.at[sem_idx], recv_sem=recv_sems.at[sem_idx],
            device_id=(dev,), device_id_type=pl.DeviceIdType.MESH)

    p1 = [
        rdma(cwq.at[0], cwq.at[1], 0, right),
        rdma(cwo.at[0], cwo.at[1], 1, right),
        rdma(cwq.at[0], cwq.at[2], 2, left),
        rdma(cwo.at[0], cwo.at[2], 3, left),
    ]
    for r in p1:
        r.start()

    kc.wait()
    vc.wait()

    for gg in range(HG):
        for b in range(B_LOC):
            k_mrg[gg, b] = k_raw[b, :, 4 * gg:4 * gg + 4, :].reshape(
                SKV, GW).astype(jnp.bfloat16)
            v_mrg[gg, b] = v_raw[b, :, 4 * gg:4 * gg + 4, :].reshape(
                SKV, GW).astype(jnp.bfloat16)

    x2b = x_ref[...].reshape(B_LOC * SQ, D_MODEL).astype(jnp.bfloat16)
    mask = _block_mask()

    def slot_contrib(slot, g):
        wq_g = cwq[slot]
        wo_g = cwo[slot]
        q = jnp.dot(x2b, wq_g, preferred_element_type=jnp.float32)
        qb = q.astype(jnp.bfloat16)
        ctx_rows = []
        for b in range(B_LOC):
            kg = k_mrg[g, b]
            vg = v_mrg[g, b]
            heads = []
            for hh in range(HG):
                qbh = qb[b * SQ:(b + 1) * SQ, hh * DH:(hh + 1) * DH]
                kbh = kg[:, hh * DH:(hh + 1) * DH]
                vbh = vg[:, hh * DH:(hh + 1) * DH]
                s = lax.dot_general(
                    qbh, kbh, (((1,), (1,)), ((), ())),
                    preferred_element_type=jnp.float32) * 0.125
                s = jnp.where(mask, s, -1e9)
                w = jnp.exp(s - jnp.max(s, axis=-1, keepdims=True))
                w = w / jnp.sum(w, axis=-1, keepdims=True)
                heads.append(jnp.dot(w.astype(jnp.bfloat16), vbh,
                                     preferred_element_type=jnp.float32))
            ctx_rows.append(jnp.concatenate(heads, axis=1))
        ctx = jnp.concatenate(ctx_rows, axis=0).astype(jnp.bfloat16)
        return jnp.dot(ctx, wo_g, preferred_element_type=jnp.float32)

    acc = slot_contrib(0, my)

    for r in p1:
        r.wait_recv()

    p2 = [
        rdma(cwq.at[1, pl.ds(0, 256)], cwq.at[3, pl.ds(0, 256)], 4, right),
        rdma(cwo.at[1, pl.ds(0, 128)], cwo.at[3, pl.ds(0, 128)], 5, right),
        rdma(cwq.at[2, pl.ds(256, 256)], cwq.at[3, pl.ds(256, 256)], 6, left),
        rdma(cwo.at[2, pl.ds(128, 128)], cwo.at[3, pl.ds(128, 128)], 7, left),
    ]
    for r in p2:
        r.start()

    acc = acc + slot_contrib(1, lax.rem(my + N_DEV - 1, N_DEV))
    acc = acc + slot_contrib(2, lax.rem(my + 1, N_DEV))

    for r in p2:
        r.wait_recv()
    acc = acc + slot_contrib(3, lax.rem(my + 2, N_DEV))

    for r in p1 + p2:
        r.wait_send()

    out_ref[...] = acc.reshape(B_LOC, SQ, D_MODEL)


def kernel(x, Wq, K_ext, V_ext, Wo):
    return pl.pallas_call(
        _body,
        out_shape=jax.ShapeDtypeStruct((B_LOC, SQ, D_MODEL), jnp.float32),
        in_specs=[
            pl.BlockSpec(memory_space=pltpu.MemorySpace.VMEM),
            pl.BlockSpec(memory_space=pltpu.MemorySpace.VMEM),
            pl.BlockSpec(memory_space=pl.ANY),
            pl.BlockSpec(memory_space=pl.ANY),
            pl.BlockSpec(memory_space=pltpu.MemorySpace.VMEM),
        ],
        out_specs=pl.BlockSpec(memory_space=pltpu.MemorySpace.VMEM),
        scratch_shapes=[
            pltpu.VMEM((B_LOC, SKV, 16, DH), jnp.float32),
            pltpu.VMEM((B_LOC, SKV, 16, DH), jnp.float32),
            pltpu.VMEM((HG, B_LOC, SKV, GW), jnp.bfloat16),
            pltpu.VMEM((HG, B_LOC, SKV, GW), jnp.bfloat16),
            pltpu.VMEM((N_DEV, D_MODEL, GW), jnp.bfloat16),
            pltpu.VMEM((N_DEV, GW, D_MODEL), jnp.bfloat16),
            pltpu.SemaphoreType.DMA((2,)),
            pltpu.SemaphoreType.DMA((8,)),
            pltpu.SemaphoreType.DMA((8,)),
        ],
        compiler_params=pltpu.CompilerParams(collective_id=0),
    )(x, Wq, K_ext, V_ext, Wo)


# device time: 25357 ns/iter; 1.2679x vs baseline; 1.1753x over previous
import jax
import jax.numpy as jnp
from jax import lax
from jax.experimental import pallas as pl
from jax.experimental.pallas import tpu as pltpu

N_DEV = 4
B_LOC = 2
SQ = 128
SKV = 128
HG = 4
GW = 4 * 64
D_MODEL = 512
DH = 64


def _block_mask():
    qi = lax.broadcasted_iota(jnp.int32, (SQ, SKV), 0) // 64
    kj = lax.broadcasted_iota(jnp.int32, (SQ, SKV), 1) // 64
    return (qi == kj) | (kj == 0) | (((qi + kj) % 3) == 0)


def _body(x_ref, wq_ref, k_ref, v_ref, wo_ref, out_ref,
          cwq, cwo, send_sems, recv_sems):
    my = lax.axis_index("i")
    left = lax.rem(my + N_DEV - 1, N_DEV)
    right = lax.rem(my + 1, N_DEV)

    barrier = pltpu.get_barrier_semaphore()
    for nbr in (left, right):
        pl.semaphore_signal(barrier, inc=1, device_id=(nbr,),
                            device_id_type=pl.DeviceIdType.MESH)
    pl.semaphore_wait(barrier, 2)

    cwq[0] = wq_ref[...]
    cwo[0] = wo_ref[...]

    def rdma(src, dst, sem_idx, dev):
        return pltpu.make_async_remote_copy(
            src_ref=src, dst_ref=dst,
            send_sem=send_sems.at[sem_idx], recv_sem=recv_sems.at[sem_idx],
            device_id=(dev,), device_id_type=pl.DeviceIdType.MESH)

    p1 = [
        rdma(cwq.at[0], cwq.at[1], 0, right),
        rdma(cwo.at[0], cwo.at[1], 1, right),
        rdma(cwq.at[0], cwq.at[2], 2, left),
        rdma(cwo.at[0], cwo.at[2], 3, left),
    ]
    for r in p1:
        r.start()

    mask = _block_mask()

    def slot_contrib(slot, g):
        wq_g = cwq[slot]
        wo_g = cwo[slot]
        q = jnp.dot(x_ref[...], wq_g, preferred_element_type=jnp.float32)
        qb = q.astype(jnp.bfloat16)
        ctx_rows = []
        for b in range(B_LOC):
            kg = k_ref[g, b]
            vg = v_ref[g, b]
            heads = []
            for hh in range(HG):
                qbh = qb[b * SQ:(b + 1) * SQ, hh * DH:(hh + 1) * DH]
                kbh = kg[:, hh * DH:(hh + 1) * DH]
                vbh = vg[:, hh * DH:(hh + 1) * DH]
                s = lax.dot_general(
                    qbh, kbh, (((1,), (1,)), ((), ())),
                    preferred_element_type=jnp.float32) * 0.125
                s = jnp.where(mask, s, -1e9)
                w = jnp.exp(s - jnp.max(s, axis=-1, keepdims=True))
                w = w / jnp.sum(w, axis=-1, keepdims=True)
                heads.append(jnp.dot(w.astype(jnp.bfloat16), vbh,
                                     preferred_element_type=jnp.float32))
            ctx_rows.append(jnp.concatenate(heads, axis=1))
        ctx = jnp.concatenate(ctx_rows, axis=0).astype(jnp.bfloat16)
        return jnp.dot(ctx, wo_g, preferred_element_type=jnp.float32)

    acc = slot_contrib(0, my)

    for r in p1:
        r.wait_recv()

    p2 = [
        rdma(cwq.at[1, pl.ds(0, 256)], cwq.at[3, pl.ds(0, 256)], 4, right),
        rdma(cwo.at[1, pl.ds(0, 128)], cwo.at[3, pl.ds(0, 128)], 5, right),
        rdma(cwq.at[2, pl.ds(256, 256)], cwq.at[3, pl.ds(256, 256)], 6, left),
        rdma(cwo.at[2, pl.ds(128, 128)], cwo.at[3, pl.ds(128, 128)], 7, left),
    ]
    for r in p2:
        r.start()

    acc = acc + slot_contrib(1, lax.rem(my + N_DEV - 1, N_DEV))
    acc = acc + slot_contrib(2, lax.rem(my + 1, N_DEV))

    for r in p2:
        r.wait_recv()
    acc = acc + slot_contrib(3, lax.rem(my + 2, N_DEV))

    for r in p1 + p2:
        r.wait_send()

    out_ref[...] = acc.reshape(B_LOC, SQ, D_MODEL)


def kernel(x, Wq, K_ext, V_ext, Wo):
    my = lax.axis_index("i")

    def prep(t):
        loc = lax.dynamic_slice_in_dim(t, my * B_LOC, B_LOC, axis=0)
        return loc.reshape(B_LOC, SKV, HG, GW).transpose(2, 0, 1, 3).astype(
            jnp.bfloat16)

    x2b = x.reshape(B_LOC * SQ, D_MODEL).astype(jnp.bfloat16)

    return pl.pallas_call(
        _body,
        out_shape=jax.ShapeDtypeStruct((B_LOC, SQ, D_MODEL), jnp.float32),
        in_specs=[pl.BlockSpec(memory_space=pltpu.MemorySpace.VMEM)] * 5,
        out_specs=pl.BlockSpec(memory_space=pltpu.MemorySpace.VMEM),
        scratch_shapes=[
            pltpu.VMEM((N_DEV, D_MODEL, GW), jnp.bfloat16),
            pltpu.VMEM((N_DEV, GW, D_MODEL), jnp.bfloat16),
            pltpu.SemaphoreType.DMA((8,)),
            pltpu.SemaphoreType.DMA((8,)),
        ],
        compiler_params=pltpu.CompilerParams(collective_id=0),
    )(x2b, Wq.astype(jnp.bfloat16), prep(K_ext), prep(V_ext),
      Wo.astype(jnp.bfloat16))


# device time: 20972 ns/iter; 1.5330x vs baseline; 1.2091x over previous
import jax
import jax.numpy as jnp
from jax import lax
from jax.experimental import pallas as pl
from jax.experimental.pallas import tpu as pltpu

N_DEV = 4
B_LOC = 2
SQ = 128
SKV = 128
HG = 4
GW = 4 * 64
D_MODEL = 512
DH = 64


def _block_mask():
    qi = lax.broadcasted_iota(jnp.int32, (SQ, SKV), 0) // 64
    kj = lax.broadcasted_iota(jnp.int32, (SQ, SKV), 1) // 64
    return (qi == kj) | (kj == 0) | (((qi + kj) % 3) == 0)


def _body(x_ref, wq_ref, k_ref, v_ref, wo_ref, out_ref,
          cwq, cwo, send_sems, recv_sems):
    my = lax.axis_index("i")
    left = lax.rem(my + N_DEV - 1, N_DEV)
    right = lax.rem(my + 1, N_DEV)

    barrier = pltpu.get_barrier_semaphore()
    for nbr in (left, right):
        pl.semaphore_signal(barrier, inc=1, device_id=(nbr,),
                            device_id_type=pl.DeviceIdType.MESH)
    pl.semaphore_wait(barrier, 2)

    cwq[0] = wq_ref[...].astype(jnp.bfloat16)
    cwo[0] = wo_ref[...].astype(jnp.bfloat16)

    def rdma(src, dst, sem_idx, dev):
        return pltpu.make_async_remote_copy(
            src_ref=src, dst_ref=dst,
            send_sem=send_sems.at[sem_idx], recv_sem=recv_sems.at[sem_idx],
            device_id=(dev,), device_id_type=pl.DeviceIdType.MESH)

    p1 = [
        rdma(cwq.at[0], cwq.at[1], 0, right),
        rdma(cwo.at[0], cwo.at[1], 1, right),
        rdma(cwq.at[0], cwq.at[2], 2, left),
        rdma(cwo.at[0], cwo.at[2], 3, left),
    ]
    for r in p1:
        r.start()

    x2b = x_ref[...].reshape(B_LOC * SQ, D_MODEL).astype(jnp.bfloat16)
    mask = _block_mask()

    def slot_contrib(slot, g):
        wq_g = cwq[slot]
        wo_g = cwo[slot]
        q = jnp.dot(x2b, wq_g, preferred_element_type=jnp.float32)
        qb = q.astype(jnp.bfloat16)
        ctx_rows = []
        for b in range(B_LOC):
            kg = k_ref[g, b]
            vg = v_ref[g, b]
            heads = []
            for hh in range(HG):
                qbh = qb[b * SQ:(b + 1) * SQ, hh * DH:(hh + 1) * DH]
                kbh = kg[:, hh * DH:(hh + 1) * DH]
                vbh = vg[:, hh * DH:(hh + 1) * DH]
                s = lax.dot_general(
                    qbh, kbh, (((1,), (1,)), ((), ())),
                    preferred_element_type=jnp.float32) * 0.125
                s = jnp.where(mask, s, -1e9)
                w = jnp.exp(s - jnp.max(s, axis=-1, keepdims=True))
                w = w / jnp.sum(w, axis=-1, keepdims=True)
                heads.append(jnp.dot(w.astype(jnp.bfloat16), vbh,
                                     preferred_element_type=jnp.float32))
            ctx_rows.append(jnp.concatenate(heads, axis=1))
        ctx = jnp.concatenate(ctx_rows, axis=0).astype(jnp.bfloat16)
        return jnp.dot(ctx, wo_g, preferred_element_type=jnp.float32)

    acc = slot_contrib(0, my)

    for r in p1:
        r.wait_recv()

    p2 = [
        rdma(cwq.at[1, pl.ds(0, 256)], cwq.at[3, pl.ds(0, 256)], 4, right),
        rdma(cwo.at[1, pl.ds(0, 128)], cwo.at[3, pl.ds(0, 128)], 5, right),
        rdma(cwq.at[2, pl.ds(256, 256)], cwq.at[3, pl.ds(256, 256)], 6, left),
        rdma(cwo.at[2, pl.ds(128, 128)], cwo.at[3, pl.ds(128, 128)], 7, left),
    ]
    for r in p2:
        r.start()

    acc = acc + slot_contrib(1, lax.rem(my + N_DEV - 1, N_DEV))
    acc = acc + slot_contrib(2, lax.rem(my + 1, N_DEV))

    for r in p2:
        r.wait_recv()
    acc = acc + slot_contrib(3, lax.rem(my + 2, N_DEV))

    for r in p1 + p2:
        r.wait_send()

    out_ref[...] = acc.reshape(B_LOC, SQ, D_MODEL)


def kernel(x, Wq, K_ext, V_ext, Wo):
    my = lax.axis_index("i")

    def prep(t):
        loc = lax.dynamic_slice_in_dim(t, my * B_LOC, B_LOC, axis=0)
        return loc.reshape(B_LOC, SKV, HG, GW).transpose(2, 0, 1, 3).astype(
            jnp.bfloat16)

    return pl.pallas_call(
        _body,
        out_shape=jax.ShapeDtypeStruct((B_LOC, SQ, D_MODEL), jnp.float32),
        in_specs=[pl.BlockSpec(memory_space=pltpu.MemorySpace.VMEM)] * 5,
        out_specs=pl.BlockSpec(memory_space=pltpu.MemorySpace.VMEM),
        scratch_shapes=[
            pltpu.VMEM((N_DEV, D_MODEL, GW), jnp.bfloat16),
            pltpu.VMEM((N_DEV, GW, D_MODEL), jnp.bfloat16),
            pltpu.SemaphoreType.DMA((8,)),
            pltpu.SemaphoreType.DMA((8,)),
        ],
        compiler_params=pltpu.CompilerParams(collective_id=0),
    )(x, Wq, prep(K_ext), prep(V_ext), Wo)


# device time: 19131 ns/iter; 1.6806x vs baseline; 1.0962x over previous
import jax
import jax.numpy as jnp
from jax import lax
from jax.experimental import pallas as pl
from jax.experimental.pallas import tpu as pltpu

N_DEV = 4
B_LOC = 2
SQ = 128
SKV = 128
HG = 4
GW = 4 * 64
D_MODEL = 512
DH = 64


def _block_mask():
    qi = lax.broadcasted_iota(jnp.int32, (SQ, SKV), 0) // 64
    kj = lax.broadcasted_iota(jnp.int32, (SQ, SKV), 1) // 64
    return (qi == kj) | (kj == 0) | (((qi + kj) % 3) == 0)


def _body(x_ref, wq_ref, k_ref, v_ref, wo_ref, out_ref,
          cwq, cwo, send_sems, recv_sems):
    my = lax.axis_index("i")
    left = lax.rem(my + N_DEV - 1, N_DEV)
    right = lax.rem(my + 1, N_DEV)

    barrier = pltpu.get_barrier_semaphore()
    for nbr in (left, right):
        pl.semaphore_signal(barrier, inc=1, device_id=(nbr,),
                            device_id_type=pl.DeviceIdType.MESH)
    pl.semaphore_wait(barrier, 2)

    cwq[0] = wq_ref[...].astype(jnp.bfloat16)
    cwo[0] = wo_ref[...].astype(jnp.bfloat16)

    def rdma(src, dst, sem_idx, dev):
        return pltpu.make_async_remote_copy(
            src_ref=src, dst_ref=dst,
            send_sem=send_sems.at[sem_idx], recv_sem=recv_sems.at[sem_idx],
            device_id=(dev,), device_id_type=pl.DeviceIdType.MESH)

    p1_wq_r = rdma(cwq.at[0], cwq.at[1], 0, right)
    p1_wq_l = rdma(cwq.at[0], cwq.at[2], 2, left)
    p1_wo_r = rdma(cwo.at[0], cwo.at[1], 1, right)
    p1_wo_l = rdma(cwo.at[0], cwo.at[2], 3, left)
    for r in (p1_wq_r, p1_wq_l, p1_wo_r, p1_wo_l):
        r.start()

    x2b = (x_ref[...].reshape(B_LOC * SQ, D_MODEL) * 0.125).astype(
        jnp.bfloat16)
    bias = jnp.where(_block_mask(), 0.0, -1e9).astype(jnp.float32)

    def attn(slot, g):
        q = jnp.dot(x2b, cwq[slot], preferred_element_type=jnp.float32)
        qb = q.astype(jnp.bfloat16)
        ctx_rows = []
        for b in range(B_LOC):
            kg = k_ref[g, b]
            vg = v_ref[g, b]
            heads = []
            for hh in range(HG):
                qbh = qb[b * SQ:(b + 1) * SQ, hh * DH:(hh + 1) * DH]
                kbh = kg[:, hh * DH:(hh + 1) * DH]
                vbh = vg[:, hh * DH:(hh + 1) * DH]
                s = lax.dot_general(
                    qbh, kbh, (((1,), (1,)), ((), ())),
                    preferred_element_type=jnp.float32) + bias
                w = jnp.exp(s)
                w = w / jnp.sum(w, axis=-1, keepdims=True)
                heads.append(jnp.dot(w.astype(jnp.bfloat16), vbh,
                                     preferred_element_type=jnp.float32))
            ctx_rows.append(jnp.concatenate(heads, axis=1))
        return jnp.concatenate(ctx_rows, axis=0).astype(jnp.bfloat16)

    def outproj(ctx, slot):
        return jnp.dot(ctx, cwo[slot], preferred_element_type=jnp.float32)

    acc = outproj(attn(0, my), 0)

    p1_wq_r.wait_recv()
    p2_wq_r = rdma(cwq.at[1, pl.ds(0, 256)], cwq.at[3, pl.ds(0, 256)],
                   4, right)
    p2_wq_r.start()
    p1_wq_l.wait_recv()
    p2_wq_l = rdma(cwq.at[2, pl.ds(256, 256)], cwq.at[3, pl.ds(256, 256)],
                   6, left)
    p2_wq_l.start()

    ctx1 = attn(1, lax.rem(my + N_DEV - 1, N_DEV))
    p1_wo_r.wait_recv()
    p2_wo_r = rdma(cwo.at[1, pl.ds(0, 128)], cwo.at[3, pl.ds(0, 128)],
                   5, right)
    p2_wo_r.start()
    acc = acc + outproj(ctx1, 1)

    ctx2 = attn(2, lax.rem(my + 1, N_DEV))
    p1_wo_l.wait_recv()
    p2_wo_l = rdma(cwo.at[2, pl.ds(128, 128)], cwo.at[3, pl.ds(128, 128)],
                   7, left)
    p2_wo_l.start()
    acc = acc + outproj(ctx2, 2)

    p2_wq_r.wait_recv()
    p2_wq_l.wait_recv()
    ctx3 = attn(3, lax.rem(my + 2, N_DEV))
    p2_wo_r.wait_recv()
    p2_wo_l.wait_recv()
    acc = acc + outproj(ctx3, 3)

    for r in (p1_wq_r, p1_wq_l, p1_wo_r, p1_wo_l,
              p2_wq_r, p2_wq_l, p2_wo_r, p2_wo_l):
        r.wait_send()

    out_ref[...] = acc.reshape(B_LOC, SQ, D_MODEL)


def kernel(x, Wq, K_ext, V_ext, Wo):
    my = lax.axis_index("i")

    def prep(t):
        loc = lax.dynamic_slice_in_dim(t, my * B_LOC, B_LOC, axis=0)
        return loc.reshape(B_LOC, SKV, HG, GW).transpose(2, 0, 1, 3).astype(
            jnp.bfloat16)

    return pl.pallas_call(
        _body,
        out_shape=jax.ShapeDtypeStruct((B_LOC, SQ, D_MODEL), jnp.float32),
        in_specs=[pl.BlockSpec(memory_space=pltpu.MemorySpace.VMEM)] * 5,
        out_specs=pl.BlockSpec(memory_space=pltpu.MemorySpace.VMEM),
        scratch_shapes=[
            pltpu.VMEM((N_DEV, D_MODEL, GW), jnp.bfloat16),
            pltpu.VMEM((N_DEV, GW, D_MODEL), jnp.bfloat16),
            pltpu.SemaphoreType.DMA((8,)),
            pltpu.SemaphoreType.DMA((8,)),
        ],
        compiler_params=pltpu.CompilerParams(collective_id=0),
    )(x, Wq, prep(K_ext), prep(V_ext), Wo)
